# baseline (speedup 1.0000x reference)
import functools

import jax
import jax.numpy as jnp
from jax import lax
from jax.experimental import pallas as pl
from jax.experimental.pallas import tpu as pltpu
from jax.experimental.pallas import tpu_sc as plsc

NC = 2
NS = 16
LANES = 16
NBUF = 3
C = 224


def _make_kernel(N, D, K):
    pair_chunks = -(-K // (C * NS))
    N0 = min(max(int(round(pair_chunks * 9 / 14)), 1), pair_chunks - 1)
    N1 = pair_chunks - N0
    S0, S1 = N0 * C, N1 * C
    START1 = NS * S0
    KP = NS * (S0 + S1)
    VALID = K - (START1 + (NS - 1) * S1)
    TAIL = VALID - (N1 - 1) * C
    n_vec = D // LANES
    assert 0 < TAIL <= C and VALID % 8 == 0 and (S1 - VALID) % LANES == 0

    mesh = plsc.VectorSubcoreMesh(
        core_axis_name="c", subcore_axis_name="s",
        num_cores=NC, num_subcores=NS)

    @functools.partial(
        pl.kernel,
        out_type=jax.ShapeDtypeStruct((K, D), jnp.float32),
        mesh=mesh,
        scratch_types=[
            pltpu.VMEM((S0,), jnp.int32),
            pltpu.VMEM((S0,), jnp.float32),
            pltpu.VMEM((NBUF, C, D), jnp.float32),
            pltpu.VMEM((LANES,), jnp.int32),
            pltpu.SemaphoreType.DMA,
            pltpu.SemaphoreType.DMA,
            pltpu.SemaphoreType.DMA,
            pltpu.SemaphoreType.DMA,
            pltpu.SemaphoreType.DMA,
            pltpu.SemaphoreType.DMA,
        ],
    )
    def gather_scale(feat_hbm, idx_hbm, scores_hbm, ng_hbm, out_hbm,
                     idx_v, sc_v, rows_v, ng_v, g0, g1, g2, o0, o1, o2):
        cid = lax.axis_index("c")
        sid = lax.axis_index("s")
        gsem = (g0, g1, g2)
        osem = (o0, o1, o2)
        on0 = cid == 0
        base = jnp.where(on0, sid * S0, START1 + sid * S1)
        ragged = (cid == 1) & (sid == NS - 1)

        pltpu.sync_copy(ng_hbm, ng_v)
        n_groups = ng_v[...][0]

        @pl.when(on0)
        def _():
            pltpu.sync_copy(idx_hbm.at[pl.ds(base, S0)], idx_v)
            pltpu.sync_copy(scores_hbm.at[pl.ds(base, S0)], sc_v)

        @pl.when(~on0 & ~ragged)
        def _():
            pltpu.sync_copy(idx_hbm.at[pl.ds(base, S1)],
                            idx_v.at[pl.ds(0, S1)])
            pltpu.sync_copy(scores_hbm.at[pl.ds(base, S1)],
                            sc_v.at[pl.ds(0, S1)])

        @pl.when(ragged)
        def _():
            pltpu.sync_copy(idx_hbm.at[pl.ds(base, VALID)],
                            idx_v.at[pl.ds(0, VALID)])
            pltpu.sync_copy(scores_hbm.at[pl.ds(base, VALID)],
                            sc_v.at[pl.ds(0, VALID)])
            for t in range((S1 - VALID) // LANES):
                idx_v[pl.ds(VALID + t * LANES, LANES)] = \
                    jnp.zeros((LANES,), jnp.int32)

        def start_gather(k):
            b = k % NBUF
            return pltpu.async_copy(
                feat_hbm.at[idx_v.at[pl.ds(k * C, C)]],
                rows_v.at[b], gsem[b])

        def wait_out(k):
            b = k % NBUF
            pltpu.make_async_copy(
                rows_v.at[b], out_hbm.at[pl.ds(base, C)], osem[b]).wait()

        def scale_chunk(k):
            b = k % NBUF

            @pl.loop(0, n_groups)
            def _(g):
                sv = sc_v[pl.ds(k * C + g * LANES, LANES)]
                for i in range(LANES):
                    s = sv[i]
                    for j in range(n_vec):
                        sl = pl.ds(j * LANES, LANES)
                        rows_v[b, g * LANES + i, sl] = \
                            rows_v[b, g * LANES + i, sl] * s

        def chunk_body(k):
            off = base + k * C
            b = k % NBUF
            nxt = k + NBUF - 1
            if nxt < N1:
                if k >= 1:
                    wait_out(k - 1)
                start_gather(nxt)
            elif nxt < N0:
                @pl.when(on0)
                def _():
                    if k >= 1:
                        wait_out(k - 1)
                    start_gather(nxt)
            pltpu.make_async_copy(
                feat_hbm.at[idx_v.at[pl.ds(k * C, C)]],
                rows_v.at[b], gsem[b]).wait()
            scale_chunk(k)
            if k == N1 - 1:
                @pl.when(~ragged)
                def _():
                    pltpu.async_copy(
                        rows_v.at[b], out_hbm.at[pl.ds(off, C)], osem[b])

                @pl.when(ragged)
                def _():
                    pltpu.async_copy(
                        rows_v.at[b].at[pl.ds(0, TAIL)],
                        out_hbm.at[pl.ds(off, TAIL)], osem[b]).wait()
            else:
                pltpu.async_copy(
                    rows_v.at[b], out_hbm.at[pl.ds(off, C)], osem[b])

        for k in range(NBUF - 1):
            if k < N1:
                start_gather(k)
            else:
                @pl.when(on0)
                def _():
                    start_gather(k)

        for k in range(N0):
            if k < N1:
                chunk_body(k)
            else:
                @pl.when(on0)
                def _():
                    chunk_body(k)

        @pl.when(on0)
        def _():
            for k in range(max(0, N0 - NBUF), N0):
                wait_out(k)

        @pl.when(~on0 & ~ragged)
        def _():
            for k in range(max(0, N1 - NBUF), N1):
                wait_out(k)

        @pl.when(ragged)
        def _():
            for k in range(max(0, N1 - NBUF), N1 - 1):
                wait_out(k)

    return gather_scale, C


def kernel(feat, select_idx, scores):
    N, D = feat.shape
    K = select_idx.shape[0]
    fn, chunk = _make_kernel(N, D, K)
    ng = jnp.full((LANES,), chunk // LANES, jnp.int32)
    return fn(feat, select_idx.astype(jnp.int32), scores, ng)


# speedup vs baseline: 1.0372x; 1.0372x over previous
import functools

import jax
import jax.numpy as jnp
from jax import lax
from jax.experimental import pallas as pl
from jax.experimental.pallas import tpu as pltpu
from jax.experimental.pallas import tpu_sc as plsc

NC = 2
NS = 16
LANES = 16
NBUF = 3
C = 224


def _make_kernel(N, D, K):
    pair_chunks = -(-K // (C * NS))
    N0 = min(max(int(round(pair_chunks * 7 / 14)), 1), pair_chunks - 1)
    N1 = pair_chunks - N0
    S0, S1 = N0 * C, N1 * C
    START1 = NS * S0
    KP = NS * (S0 + S1)
    VALID = K - (START1 + (NS - 1) * S1)
    TAIL = VALID - (N1 - 1) * C
    n_vec = D // LANES
    assert 0 < TAIL <= C and VALID % 8 == 0 and (S1 - VALID) % LANES == 0

    mesh = plsc.VectorSubcoreMesh(
        core_axis_name="c", subcore_axis_name="s",
        num_cores=NC, num_subcores=NS)

    @functools.partial(
        pl.kernel,
        out_type=jax.ShapeDtypeStruct((K, D), jnp.float32),
        mesh=mesh,
        scratch_types=[
            pltpu.VMEM((S0,), jnp.int32),
            pltpu.VMEM((S0,), jnp.float32),
            pltpu.VMEM((NBUF, C, D), jnp.float32),
            pltpu.VMEM((LANES,), jnp.int32),
            pltpu.SemaphoreType.DMA,
            pltpu.SemaphoreType.DMA,
            pltpu.SemaphoreType.DMA,
            pltpu.SemaphoreType.DMA,
            pltpu.SemaphoreType.DMA,
            pltpu.SemaphoreType.DMA,
        ],
    )
    def gather_scale(feat_hbm, idx_hbm, scores_hbm, ng_hbm, out_hbm,
                     idx_v, sc_v, rows_v, ng_v, g0, g1, g2, o0, o1, o2):
        cid = lax.axis_index("c")
        sid = lax.axis_index("s")
        gsem = (g0, g1, g2)
        osem = (o0, o1, o2)
        on0 = cid == 0
        base = jnp.where(on0, sid * S0, START1 + sid * S1)
        ragged = (cid == 1) & (sid == NS - 1)

        pltpu.sync_copy(ng_hbm, ng_v)
        n_groups = ng_v[...][0]

        @pl.when(on0)
        def _():
            pltpu.sync_copy(idx_hbm.at[pl.ds(base, S0)], idx_v)
            pltpu.sync_copy(scores_hbm.at[pl.ds(base, S0)], sc_v)

        @pl.when(~on0 & ~ragged)
        def _():
            pltpu.sync_copy(idx_hbm.at[pl.ds(base, S1)],
                            idx_v.at[pl.ds(0, S1)])
            pltpu.sync_copy(scores_hbm.at[pl.ds(base, S1)],
                            sc_v.at[pl.ds(0, S1)])

        @pl.when(ragged)
        def _():
            pltpu.sync_copy(idx_hbm.at[pl.ds(base, VALID)],
                            idx_v.at[pl.ds(0, VALID)])
            pltpu.sync_copy(scores_hbm.at[pl.ds(base, VALID)],
                            sc_v.at[pl.ds(0, VALID)])
            for t in range((S1 - VALID) // LANES):
                idx_v[pl.ds(VALID + t * LANES, LANES)] = \
                    jnp.zeros((LANES,), jnp.int32)

        SPLIT = 4
        H = C // SPLIT

        def start_gather(k):
            b = k % NBUF
            for p in range(SPLIT):
                pltpu.async_copy(
                    feat_hbm.at[idx_v.at[pl.ds(k * C + p * H, H)]],
                    rows_v.at[b].at[pl.ds(p * H, H)], gsem[b])

        def wait_gather(k):
            b = k % NBUF
            for p in range(SPLIT):
                pltpu.make_async_copy(
                    feat_hbm.at[idx_v.at[pl.ds(k * C + p * H, H)]],
                    rows_v.at[b].at[pl.ds(p * H, H)], gsem[b]).wait()

        def start_out(k, off):
            b = k % NBUF
            for p in range(SPLIT):
                pltpu.async_copy(
                    rows_v.at[b].at[pl.ds(p * H, H)],
                    out_hbm.at[pl.ds(off + p * H, H)], osem[b])

        def wait_out(k):
            b = k % NBUF
            for p in range(SPLIT):
                pltpu.make_async_copy(
                    rows_v.at[b].at[pl.ds(p * H, H)],
                    out_hbm.at[pl.ds(base, H)], osem[b]).wait()

        def scale_chunk(k):
            b = k % NBUF

            @pl.loop(0, n_groups)
            def _(g):
                sv = sc_v[pl.ds(k * C + g * LANES, LANES)]
                for i in range(LANES):
                    s = sv[i]
                    for j in range(n_vec):
                        sl = pl.ds(j * LANES, LANES)
                        rows_v[b, g * LANES + i, sl] = \
                            rows_v[b, g * LANES + i, sl] * s

        def chunk_body(k):
            off = base + k * C
            b = k % NBUF
            nxt = k + NBUF - 1
            if nxt < N1:
                if k >= 1:
                    wait_out(k - 1)
                start_gather(nxt)
            elif nxt < N0:
                @pl.when(on0)
                def _():
                    if k >= 1:
                        wait_out(k - 1)
                    start_gather(nxt)
            wait_gather(k)
            scale_chunk(k)
            if k == N1 - 1:
                @pl.when(~ragged)
                def _():
                    start_out(k, off)

                @pl.when(ragged)
                def _():
                    pltpu.async_copy(
                        rows_v.at[b].at[pl.ds(0, TAIL)],
                        out_hbm.at[pl.ds(off, TAIL)], osem[b]).wait()
            else:
                start_out(k, off)

        for k in range(NBUF - 1):
            if k < N1:
                start_gather(k)
            else:
                @pl.when(on0)
                def _():
                    start_gather(k)

        for k in range(N0):
            if k < N1:
                chunk_body(k)
            else:
                @pl.when(on0)
                def _():
                    chunk_body(k)

        @pl.when(on0)
        def _():
            for k in range(max(0, N0 - NBUF), N0):
                wait_out(k)

        @pl.when(~on0 & ~ragged)
        def _():
            for k in range(max(0, N1 - NBUF), N1):
                wait_out(k)

        @pl.when(ragged)
        def _():
            for k in range(max(0, N1 - NBUF), N1 - 1):
                wait_out(k)

    return gather_scale, C


def kernel(feat, select_idx, scores):
    N, D = feat.shape
    K = select_idx.shape[0]
    fn, chunk = _make_kernel(N, D, K)
    ng = jnp.full((LANES,), chunk // LANES, jnp.int32)
    return fn(feat, select_idx.astype(jnp.int32), scores, ng)


# speedup vs baseline: 1.0516x; 1.0138x over previous
import functools

import jax
import jax.numpy as jnp
from jax import lax
from jax.experimental import pallas as pl
from jax.experimental.pallas import tpu as pltpu
from jax.experimental.pallas import tpu_sc as plsc

NC = 2
NS = 16
LANES = 16
NBUF = 3
C = 224


def _make_kernel(N, D, K):
    pair_chunks = -(-K // (C * NS))
    N0 = min(max(int(round(pair_chunks * 7 / 14)), 1), pair_chunks - 1)
    N1 = pair_chunks - N0
    S0, S1 = N0 * C, N1 * C
    START1 = NS * S0
    KP = NS * (S0 + S1)
    VALID = K - (START1 + (NS - 1) * S1)
    TAIL = VALID - (N1 - 1) * C
    n_vec = D // LANES
    assert 0 < TAIL <= C and VALID % 8 == 0 and (S1 - VALID) % LANES == 0

    mesh = plsc.VectorSubcoreMesh(
        core_axis_name="c", subcore_axis_name="s",
        num_cores=NC, num_subcores=NS)

    @functools.partial(
        pl.kernel,
        out_type=jax.ShapeDtypeStruct((K, D), jnp.float32),
        mesh=mesh,
        scratch_types=[
            pltpu.VMEM((S0,), jnp.int32),
            pltpu.VMEM((S0,), jnp.float32),
            pltpu.VMEM((NBUF, C, D), jnp.float32),
            pltpu.VMEM((LANES,), jnp.int32),
            pltpu.SemaphoreType.DMA,
            pltpu.SemaphoreType.DMA,
            pltpu.SemaphoreType.DMA,
            pltpu.SemaphoreType.DMA,
            pltpu.SemaphoreType.DMA,
            pltpu.SemaphoreType.DMA,
        ],
    )
    def gather_scale(feat_hbm, idx_hbm, scores_hbm, ng_hbm, out_hbm,
                     idx_v, sc_v, rows_v, ng_v, g0, g1, g2, o0, o1, o2):
        cid = lax.axis_index("c")
        sid = lax.axis_index("s")
        gsem = (g0, g1, g2)
        osem = (o0, o1, o2)
        on0 = cid == 0
        base = pl.multiple_of(
            jnp.where(on0, sid * S0, START1 + sid * S1), 32)
        ragged = (cid == 1) & (sid == NS - 1)

        pltpu.sync_copy(ng_hbm, ng_v)
        n_groups = ng_v[...][0]

        @pl.when(on0)
        def _():
            pltpu.sync_copy(idx_hbm.at[pl.ds(base, S0)], idx_v)
            pltpu.sync_copy(scores_hbm.at[pl.ds(base, S0)], sc_v)

        @pl.when(~on0 & ~ragged)
        def _():
            pltpu.sync_copy(idx_hbm.at[pl.ds(base, S1)],
                            idx_v.at[pl.ds(0, S1)])
            pltpu.sync_copy(scores_hbm.at[pl.ds(base, S1)],
                            sc_v.at[pl.ds(0, S1)])

        @pl.when(ragged)
        def _():
            pltpu.sync_copy(idx_hbm.at[pl.ds(base, VALID)],
                            idx_v.at[pl.ds(0, VALID)])
            pltpu.sync_copy(scores_hbm.at[pl.ds(base, VALID)],
                            sc_v.at[pl.ds(0, VALID)])
            for t in range((S1 - VALID) // LANES):
                idx_v[pl.ds(VALID + t * LANES, LANES)] = \
                    jnp.zeros((LANES,), jnp.int32)

        SPLIT = 1
        H = C // SPLIT

        def start_gather(k):
            b = k % NBUF
            for p in range(SPLIT):
                pltpu.async_copy(
                    feat_hbm.at[idx_v.at[pl.ds(k * C + p * H, H)]],
                    rows_v.at[b].at[pl.ds(p * H, H)], gsem[b])

        def wait_gather(k):
            b = k % NBUF
            for p in range(SPLIT):
                pltpu.make_async_copy(
                    feat_hbm.at[idx_v.at[pl.ds(k * C + p * H, H)]],
                    rows_v.at[b].at[pl.ds(p * H, H)], gsem[b]).wait()

        def start_out(k, off):
            b = k % NBUF
            for p in range(SPLIT):
                pltpu.async_copy(
                    rows_v.at[b].at[pl.ds(p * H, H)],
                    out_hbm.at[pl.ds(off + p * H, H)], osem[b])

        def wait_out(k):
            b = k % NBUF
            for p in range(SPLIT):
                pltpu.make_async_copy(
                    rows_v.at[b].at[pl.ds(p * H, H)],
                    out_hbm.at[pl.ds(base, H)], osem[b]).wait()

        def scale_chunk(k):
            b = k % NBUF

            @pl.loop(0, n_groups)
            def _(g):
                sv = sc_v[pl.ds(k * C + g * LANES, LANES)]
                for i in range(LANES):
                    s = sv[i]
                    for j in range(n_vec):
                        sl = pl.ds(j * LANES, LANES)
                        rows_v[b, g * LANES + i, sl] = \
                            rows_v[b, g * LANES + i, sl] * s

        def chunk_body(k):
            off = pl.multiple_of(base + k * C, 32)
            b = k % NBUF
            nxt = k + NBUF - 1
            if nxt < N1:
                if k >= 1:
                    wait_out(k - 1)
                start_gather(nxt)
            elif nxt < N0:
                @pl.when(on0)
                def _():
                    if k >= 1:
                        wait_out(k - 1)
                    start_gather(nxt)
            wait_gather(k)
            scale_chunk(k)
            if k == N1 - 1:
                @pl.when(~ragged)
                def _():
                    start_out(k, off)

                @pl.when(ragged)
                def _():
                    pltpu.async_copy(
                        rows_v.at[b].at[pl.ds(0, TAIL)],
                        out_hbm.at[pl.ds(off, TAIL)], osem[b]).wait()
            else:
                start_out(k, off)

        for k in range(NBUF - 1):
            if k < N1:
                start_gather(k)
            else:
                @pl.when(on0)
                def _():
                    start_gather(k)

        for k in range(N0):
            if k < N1:
                chunk_body(k)
            else:
                @pl.when(on0)
                def _():
                    chunk_body(k)

        @pl.when(on0)
        def _():
            for k in range(max(0, N0 - NBUF), N0):
                wait_out(k)

        @pl.when(~on0 & ~ragged)
        def _():
            for k in range(max(0, N1 - NBUF), N1):
                wait_out(k)

        @pl.when(ragged)
        def _():
            for k in range(max(0, N1 - NBUF), N1 - 1):
                wait_out(k)

    return gather_scale, C


def kernel(feat, select_idx, scores):
    N, D = feat.shape
    K = select_idx.shape[0]
    fn, chunk = _make_kernel(N, D, K)
    ng = jnp.full((LANES,), chunk // LANES, jnp.int32)
    return fn(feat, select_idx.astype(jnp.int32), scores, ng)
